# Initial kernel scaffold; baseline (speedup 1.0000x reference)
import jax
import jax.numpy as jnp
from jax.experimental import pallas as pl


def kernel(inputs, embedding_weight):
    raise NotImplementedError("write your pallas kernel here")


# speedup vs baseline: 1.0776x; 1.0776x over previous
import functools

import jax
import jax.numpy as jnp
from jax import lax
from jax.experimental import pallas as pl
from jax.experimental.pallas import tpu as pltpu
from jax.experimental.pallas import tpu_sc as plsc

LATENT = 1024
WORD = 256
NB = 8192
BATCH = 4096
M = BATCH * LATENT // WORD
COMMIT = 2.5

BM = 512
BN = 1024
GRID_M = M // BM
N_CHUNKS = NB // BN


def _dist_argmin_kernel(z_ref, e_ref, z2_ref, e2_ref, idx_ref, loss_ref,
                        acc_ref):
    pid = pl.program_id(0)

    @pl.when(pid == 0)
    def _init():
        acc_ref[0] = 0.0

    z = z_ref[...]
    z2 = z2_ref[...]

    zh = z.astype(jnp.bfloat16)

    def body(c, carry):
        minv, mini = carry
        e_blk = e_ref[pl.ds(c * BN, BN), :]
        e2_blk = e2_ref[0, pl.ds(c * BN, BN)]
        mm = lax.dot_general(zh, e_blk.astype(jnp.bfloat16),
                             (((1,), (1,)), ((), ())),
                             preferred_element_type=jnp.float32)
        d = (z2 + e2_blk[None, :]) - 2.0 * mm
        lmin = jnp.min(d, axis=1)
        iota = lax.broadcasted_iota(jnp.int32, (BM, BN), 1)
        larg = jnp.min(jnp.where(d == lmin[:, None], iota, NB),
                       axis=1) + c * BN
        take = lmin < minv
        return jnp.where(take, lmin, minv), jnp.where(take, larg, mini)

    init = (jnp.full((BM,), jnp.inf, jnp.float32), jnp.zeros((BM,), jnp.int32))
    minv, mini = lax.fori_loop(0, N_CHUNKS, body, init)

    idx_ref[...] = mini.reshape(1, 1, BM)
    acc_ref[0] += jnp.sum(minv)

    @pl.when(pid == GRID_M - 1)
    def _fin():
        loss_ref[0] = acc_ref[0] * ((1.0 + COMMIT) / (BATCH * LATENT))


def _dist_argmin(z, emb, z2, e2):
    return pl.pallas_call(
        _dist_argmin_kernel,
        grid=(GRID_M,),
        in_specs=[
            pl.BlockSpec((BM, WORD), lambda i: (i, 0)),
            pl.BlockSpec((NB, WORD), lambda i: (0, 0)),
            pl.BlockSpec((BM, 1), lambda i: (i, 0)),
            pl.BlockSpec((1, NB), lambda i: (0, 0)),
        ],
        out_specs=[
            pl.BlockSpec((1, 1, BM), lambda i: (i, 0, 0)),
            pl.BlockSpec(memory_space=pltpu.SMEM, block_shape=(1,),
                         index_map=lambda i: (0,)),
        ],
        out_shape=[
            jax.ShapeDtypeStruct((GRID_M, 1, BM), jnp.int32),
            jax.ShapeDtypeStruct((1,), jnp.float32),
        ],
        scratch_shapes=[
            pltpu.SMEM((1,), jnp.float32),
        ],
    )(z, emb, z2, e2)


_SC_INFO = plsc.get_sparse_core_info()
_NC, _NS = _SC_INFO.num_cores, _SC_INFO.num_subcores
NW = _NC * _NS
BPW = M // NW
CH = 128
NCH = BPW // CH
NBUF = 2


def _gather_body(table_hbm, idx_hbm, out_hbm, idx_v, bufs, sems):
    wid = lax.axis_index("s") * _NC + lax.axis_index("c")
    base = wid * BPW
    pltpu.sync_copy(idx_hbm.at[pl.ds(base, BPW)], idx_v)

    copies = [None] * NBUF
    for b in range(NBUF):
        copies[b] = pltpu.async_copy(
            table_hbm.at[idx_v.at[pl.ds(b * CH, CH)]], bufs.at[b], sems.at[b])
    for c in range(NCH):
        s = c % NBUF
        copies[s].wait()
        pltpu.sync_copy(bufs.at[s], out_hbm.at[pl.ds(base + c * CH, CH)])
        nxt = c + NBUF
        if nxt < NCH:
            copies[s] = pltpu.async_copy(
                table_hbm.at[idx_v.at[pl.ds(nxt * CH, CH)]], bufs.at[s],
                sems.at[s])


def _sc_gather(emb, idx):
    mesh = plsc.VectorSubcoreMesh(core_axis_name="c", subcore_axis_name="s")
    k = functools.partial(
        pl.kernel,
        mesh=mesh,
        out_type=jax.ShapeDtypeStruct((M, WORD), jnp.float32),
        scratch_types=[
            pltpu.VMEM((BPW,), jnp.int32),
            pltpu.VMEM((NBUF, CH, WORD), jnp.float32),
            pltpu.SemaphoreType.DMA((NBUF,)),
        ],
    )(_gather_body)
    return k(emb, idx)


def kernel(inputs, embedding_weight):
    z_mean = inputs[0]
    z = z_mean.reshape(M, WORD)
    z2 = jnp.sum(z ** 2, axis=1, keepdims=True)
    e2 = jnp.sum(embedding_weight ** 2, axis=1).reshape(1, NB)
    idx, loss = _dist_argmin(z, embedding_weight, z2, e2)
    z_q = _sc_gather(embedding_weight, idx.reshape(M))
    return z_q.reshape(z_mean.shape), loss[0]


# speedup vs baseline: 1.1292x; 1.0479x over previous
import functools

import jax
import jax.numpy as jnp
from jax import lax
from jax.experimental import pallas as pl
from jax.experimental.pallas import tpu as pltpu
from jax.experimental.pallas import tpu_sc as plsc

LATENT = 1024
WORD = 256
NB = 8192
BATCH = 4096
M = BATCH * LATENT // WORD
COMMIT = 2.5

BM = 512
BN = 1024
GRID_M = M // BM
N_CHUNKS = NB // BN


def _dist_argmin_kernel(zh_ref, eh_ref, z2_ref, e2_ref, idx_ref, loss_ref,
                        acc_ref):
    pid = pl.program_id(0)

    @pl.when(pid == 0)
    def _init():
        acc_ref[0] = 0.0

    zh = zh_ref[...]
    z2 = z2_ref[...]
    iota = lax.broadcasted_iota(jnp.int32, (BM, BN), 1).astype(jnp.float32)

    def body(c, carry):
        minv, mini = carry
        eh_blk = eh_ref[pl.ds(c * BN, BN), :]
        e2_blk = e2_ref[0, pl.ds(c * BN, BN)]
        mm = lax.dot_general(zh, eh_blk, (((1,), (1,)), ((), ())),
                             preferred_element_type=jnp.float32)
        d = (z2 + e2_blk[None, :]) - 2.0 * mm
        lmin = jnp.min(d, axis=1)
        larg = jnp.min(jnp.where(d == lmin[:, None], iota, float(NB)),
                       axis=1) + c * float(BN)
        take = lmin < minv
        return jnp.where(take, lmin, minv), jnp.where(take, larg, mini)

    init = (jnp.full((BM,), jnp.inf, jnp.float32),
            jnp.zeros((BM,), jnp.float32))
    minv, mini = lax.fori_loop(0, N_CHUNKS, body, init)

    idx_ref[...] = mini.astype(jnp.int32).reshape(1, 1, BM)
    acc_ref[0] += jnp.sum(minv)

    @pl.when(pid == GRID_M - 1)
    def _fin():
        loss_ref[0] = acc_ref[0] * ((1.0 + COMMIT) / (BATCH * LATENT))


def _dist_argmin(zh, eh, z2, e2):
    return pl.pallas_call(
        _dist_argmin_kernel,
        grid=(GRID_M,),
        in_specs=[
            pl.BlockSpec((BM, WORD), lambda i: (i, 0)),
            pl.BlockSpec((NB, WORD), lambda i: (0, 0)),
            pl.BlockSpec((BM, 1), lambda i: (i, 0)),
            pl.BlockSpec((1, NB), lambda i: (0, 0)),
        ],
        out_specs=[
            pl.BlockSpec((1, 1, BM), lambda i: (i, 0, 0)),
            pl.BlockSpec(memory_space=pltpu.SMEM, block_shape=(1,),
                         index_map=lambda i: (0,)),
        ],
        out_shape=[
            jax.ShapeDtypeStruct((GRID_M, 1, BM), jnp.int32),
            jax.ShapeDtypeStruct((1,), jnp.float32),
        ],
        scratch_shapes=[
            pltpu.SMEM((1,), jnp.float32),
        ],
    )(zh, eh, z2, e2)


_SC_INFO = plsc.get_sparse_core_info()
_NC, _NS = _SC_INFO.num_cores, _SC_INFO.num_subcores
NW = _NC * _NS
BPW = M // NW
CH = 128
NCH = BPW // CH
NBUF = 2


def _gather_body(table_hbm, idx_hbm, out_hbm, idx_v, bufs, sems):
    wid = lax.axis_index("s") * _NC + lax.axis_index("c")
    base = wid * BPW
    pltpu.sync_copy(idx_hbm.at[pl.ds(base, BPW)], idx_v)

    copies = [None] * NBUF
    for b in range(NBUF):
        copies[b] = pltpu.async_copy(
            table_hbm.at[idx_v.at[pl.ds(b * CH, CH)]], bufs.at[b], sems.at[b])
    for c in range(NCH):
        s = c % NBUF
        copies[s].wait()
        pltpu.sync_copy(bufs.at[s], out_hbm.at[pl.ds(base + c * CH, CH)])
        nxt = c + NBUF
        if nxt < NCH:
            copies[s] = pltpu.async_copy(
                table_hbm.at[idx_v.at[pl.ds(nxt * CH, CH)]], bufs.at[s],
                sems.at[s])


def _sc_gather(emb, idx):
    mesh = plsc.VectorSubcoreMesh(core_axis_name="c", subcore_axis_name="s")
    k = functools.partial(
        pl.kernel,
        mesh=mesh,
        out_type=jax.ShapeDtypeStruct((M, WORD), jnp.float32),
        scratch_types=[
            pltpu.VMEM((BPW,), jnp.int32),
            pltpu.VMEM((NBUF, CH, WORD), jnp.float32),
            pltpu.SemaphoreType.DMA((NBUF,)),
        ],
    )(_gather_body)
    return k(emb, idx)


def kernel(inputs, embedding_weight):
    z_mean = inputs[0]
    z = z_mean.reshape(M, WORD)
    z2 = jnp.sum(z ** 2, axis=1, keepdims=True)
    e2 = jnp.sum(embedding_weight ** 2, axis=1).reshape(1, NB)
    idx, loss = _dist_argmin(z.astype(jnp.bfloat16),
                             embedding_weight.astype(jnp.bfloat16), z2, e2)
    z_q = _sc_gather(embedding_weight, idx.reshape(M))
    return z_q.reshape(z_mean.shape), loss[0]


# speedup vs baseline: 1.5663x; 1.3871x over previous
import functools

import jax
import jax.numpy as jnp
from jax import lax
from jax.experimental import pallas as pl
from jax.experimental.pallas import tpu as pltpu
from jax.experimental.pallas import tpu_sc as plsc

LATENT = 1024
WORD = 256
NB = 8192
BATCH = 4096
M = BATCH * LATENT // WORD
COMMIT = 2.5

BM = 512
BN = 1024
GRID_M = M // BM
N_CHUNKS = NB // BN


def _dist_argmin_kernel(zh_ref, eh_ref, z2_ref, e2_ref, idx_ref, loss_ref,
                        acc_ref):
    pid = pl.program_id(0)

    @pl.when(pid == 0)
    def _init():
        acc_ref[0] = 0.0

    zh = zh_ref[...]
    z2b = jnp.broadcast_to(z2_ref[...], (BM, 128))
    iota = lax.broadcasted_iota(jnp.int32, (1, 128), 1).astype(jnp.float32)

    runv = jnp.full((BM, 128), jnp.inf, jnp.float32)
    runi = jnp.zeros((BM, 128), jnp.float32)
    for c in range(N_CHUNKS):
        eh_blk = eh_ref[c * BN:(c + 1) * BN, :]
        mm2 = lax.dot_general(zh, eh_blk, (((1,), (1,)), ((), ())),
                              preferred_element_type=jnp.float32)
        for v in range(BN // 128):
            lo = v * 128
            e2v = e2_ref[0, c * BN + lo:c * BN + lo + 128]
            dv = (z2b + e2v[None, :]) + mm2[:, lo:lo + 128]
            iv = iota + float(c * BN + lo)
            m = dv < runv
            runv = jnp.where(m, dv, runv)
            runi = jnp.where(m, jnp.broadcast_to(iv, (BM, 128)), runi)

    lmin = jnp.min(runv, axis=1)
    mini = jnp.min(jnp.where(runv == lmin[:, None], runi, float(NB)), axis=1)
    idx_ref[...] = mini.astype(jnp.int32).reshape(1, 1, BM)
    acc_ref[0] += jnp.sum(lmin)

    @pl.when(pid == GRID_M - 1)
    def _fin():
        loss_ref[0] = acc_ref[0] * ((1.0 + COMMIT) / (BATCH * LATENT))


def _dist_argmin(zh, eh, z2, e2):
    return pl.pallas_call(
        _dist_argmin_kernel,
        grid=(GRID_M,),
        in_specs=[
            pl.BlockSpec((BM, WORD), lambda i: (i, 0)),
            pl.BlockSpec((NB, WORD), lambda i: (0, 0)),
            pl.BlockSpec((BM, 1), lambda i: (i, 0)),
            pl.BlockSpec((1, NB), lambda i: (0, 0)),
        ],
        out_specs=[
            pl.BlockSpec((1, 1, BM), lambda i: (i, 0, 0)),
            pl.BlockSpec(memory_space=pltpu.SMEM, block_shape=(1,),
                         index_map=lambda i: (0,)),
        ],
        out_shape=[
            jax.ShapeDtypeStruct((GRID_M, 1, BM), jnp.int32),
            jax.ShapeDtypeStruct((1,), jnp.float32),
        ],
        scratch_shapes=[
            pltpu.SMEM((1,), jnp.float32),
        ],
    )(zh, eh, z2, e2)


_SC_INFO = plsc.get_sparse_core_info()
_NC, _NS = _SC_INFO.num_cores, _SC_INFO.num_subcores
NW = _NC * _NS
BPW = M // NW
CH = 128
NCH = BPW // CH
NBUF = 2


def _gather_body(table_hbm, idx_hbm, out_hbm, idx_v, bufs, sems):
    wid = lax.axis_index("s") * _NC + lax.axis_index("c")
    base = wid * BPW
    pltpu.sync_copy(idx_hbm.at[pl.ds(base, BPW)], idx_v)

    copies = [None] * NBUF
    for b in range(NBUF):
        copies[b] = pltpu.async_copy(
            table_hbm.at[idx_v.at[pl.ds(b * CH, CH)]], bufs.at[b], sems.at[b])
    for c in range(NCH):
        s = c % NBUF
        copies[s].wait()
        pltpu.sync_copy(bufs.at[s], out_hbm.at[pl.ds(base + c * CH, CH)])
        nxt = c + NBUF
        if nxt < NCH:
            copies[s] = pltpu.async_copy(
                table_hbm.at[idx_v.at[pl.ds(nxt * CH, CH)]], bufs.at[s],
                sems.at[s])


def _sc_gather(emb, idx):
    mesh = plsc.VectorSubcoreMesh(core_axis_name="c", subcore_axis_name="s")
    k = functools.partial(
        pl.kernel,
        mesh=mesh,
        out_type=jax.ShapeDtypeStruct((M, WORD), jnp.float32),
        scratch_types=[
            pltpu.VMEM((BPW,), jnp.int32),
            pltpu.VMEM((NBUF, CH, WORD), jnp.float32),
            pltpu.SemaphoreType.DMA((NBUF,)),
        ],
    )(_gather_body)
    return k(emb, idx)


def kernel(inputs, embedding_weight):
    z_mean = inputs[0]
    z = z_mean.reshape(M, WORD)
    z2 = jnp.sum(z ** 2, axis=1, keepdims=True)
    e2 = jnp.sum(embedding_weight ** 2, axis=1).reshape(1, NB)
    idx, loss = _dist_argmin(z.astype(jnp.bfloat16),
                             embedding_weight.astype(jnp.bfloat16) * -2.0, z2,
                             e2)
    z_q = _sc_gather(embedding_weight, idx.reshape(M))
    return z_q.reshape(z_mean.shape), loss[0]


# speedup vs baseline: 1.8033x; 1.1513x over previous
import functools

import jax
import jax.numpy as jnp
from jax import lax
from jax.experimental import pallas as pl
from jax.experimental.pallas import tpu as pltpu
from jax.experimental.pallas import tpu_sc as plsc

LATENT = 1024
WORD = 256
NB = 8192
BATCH = 4096
M = BATCH * LATENT // WORD
COMMIT = 2.5

BM = 1024
BN = 1024
GRID_M = M // BM
N_CHUNKS = NB // BN


def _dist_argmin_kernel(z_ref, eh_ref, e2_ref, idx_ref, loss_ref, acc_ref):
    pid = pl.program_id(0)

    @pl.when(pid == 0)
    def _init():
        acc_ref[0] = 0.0

    z = z_ref[...]
    zh = z.astype(jnp.bfloat16)
    z2b = jnp.broadcast_to(jnp.sum(z * z, axis=1, keepdims=True), (BM, 128))
    iota = lax.broadcasted_iota(jnp.int32, (1, 128), 1).astype(jnp.float32)

    runv = jnp.full((BM, 128), jnp.inf, jnp.float32)
    runi = jnp.zeros((BM, 128), jnp.float32)
    for c in range(N_CHUNKS):
        eh_blk = eh_ref[c * BN:(c + 1) * BN, :]
        mm2 = lax.dot_general(zh, eh_blk, (((1,), (1,)), ((), ())),
                              preferred_element_type=jnp.float32)
        for v in range(BN // 128):
            lo = v * 128
            e2v = e2_ref[0, c * BN + lo:c * BN + lo + 128]
            dv = (z2b + e2v[None, :]) + mm2[:, lo:lo + 128]
            iv = iota + float(c * BN + lo)
            m = dv < runv
            runv = jnp.where(m, dv, runv)
            runi = jnp.where(m, jnp.broadcast_to(iv, (BM, 128)), runi)

    lmin = jnp.min(runv, axis=1)
    mini = jnp.min(jnp.where(runv == lmin[:, None], runi, float(NB)), axis=1)
    idx_ref[...] = mini.astype(jnp.int32).reshape(1, 1, BM)
    acc_ref[0] += jnp.sum(lmin)

    @pl.when(pid == GRID_M - 1)
    def _fin():
        loss_ref[0] = acc_ref[0] * ((1.0 + COMMIT) / (BATCH * LATENT))


def _dist_argmin(z, eh, e2):
    return pl.pallas_call(
        _dist_argmin_kernel,
        grid=(GRID_M,),
        in_specs=[
            pl.BlockSpec((BM, WORD), lambda i: (i, 0)),
            pl.BlockSpec((NB, WORD), lambda i: (0, 0)),
            pl.BlockSpec((1, NB), lambda i: (0, 0)),
        ],
        out_specs=[
            pl.BlockSpec((1, 1, BM), lambda i: (i, 0, 0)),
            pl.BlockSpec(memory_space=pltpu.SMEM, block_shape=(1,),
                         index_map=lambda i: (0,)),
        ],
        out_shape=[
            jax.ShapeDtypeStruct((GRID_M, 1, BM), jnp.int32),
            jax.ShapeDtypeStruct((1,), jnp.float32),
        ],
        scratch_shapes=[
            pltpu.SMEM((1,), jnp.float32),
        ],
    )(z, eh, e2)


_SC_INFO = plsc.get_sparse_core_info()
_NC, _NS = _SC_INFO.num_cores, _SC_INFO.num_subcores
NW = _NC * _NS
BPW = M // NW
CH = 128
NCH = BPW // CH
NBUF = 2


def _gather_body(table_hbm, idx_hbm, out_hbm, idx_v, bufs, sems):
    wid = lax.axis_index("s") * _NC + lax.axis_index("c")
    base = wid * BPW
    pltpu.sync_copy(idx_hbm.at[pl.ds(base, BPW)], idx_v)

    copies = [None] * NBUF
    for b in range(NBUF):
        copies[b] = pltpu.async_copy(
            table_hbm.at[idx_v.at[pl.ds(b * CH, CH)]], bufs.at[b], sems.at[b])
    for c in range(NCH):
        s = c % NBUF
        copies[s].wait()
        pltpu.sync_copy(bufs.at[s], out_hbm.at[pl.ds(base + c * CH, CH)])
        nxt = c + NBUF
        if nxt < NCH:
            copies[s] = pltpu.async_copy(
                table_hbm.at[idx_v.at[pl.ds(nxt * CH, CH)]], bufs.at[s],
                sems.at[s])


def _sc_gather(emb, idx):
    mesh = plsc.VectorSubcoreMesh(core_axis_name="c", subcore_axis_name="s")
    k = functools.partial(
        pl.kernel,
        mesh=mesh,
        out_type=jax.ShapeDtypeStruct((M, WORD), jnp.float32),
        scratch_types=[
            pltpu.VMEM((BPW,), jnp.int32),
            pltpu.VMEM((NBUF, CH, WORD), jnp.float32),
            pltpu.SemaphoreType.DMA((NBUF,)),
        ],
    )(_gather_body)
    return k(emb, idx)


def kernel(inputs, embedding_weight):
    z_mean = inputs[0]
    z = z_mean.reshape(M, WORD)
    e2 = jnp.sum(embedding_weight ** 2, axis=1).reshape(1, NB)
    idx, loss = _dist_argmin(z, embedding_weight.astype(jnp.bfloat16) * -2.0,
                             e2)
    z_q = _sc_gather(embedding_weight, idx.reshape(M))
    return z_q.reshape(z_mean.shape), loss[0]


# speedup vs baseline: 1.8224x; 1.0106x over previous
import functools

import jax
import jax.numpy as jnp
from jax import lax
from jax.experimental import pallas as pl
from jax.experimental.pallas import tpu as pltpu
from jax.experimental.pallas import tpu_sc as plsc

LATENT = 1024
WORD = 256
NB = 8192
BATCH = 4096
M = BATCH * LATENT // WORD
COMMIT = 2.5

BM = 1024
BN = 1024
GRID_M = M // BM
N_CHUNKS = NB // BN


def _dist_argmin_kernel(z_ref, e_ref, idx_ref, loss_ref, eh_ref, e2_ref,
                        acc_ref):
    pid = pl.program_id(0)

    @pl.when(pid == 0)
    def _init():
        e = e_ref[...]
        eh_ref[...] = e.astype(jnp.bfloat16) * -2.0
        e2_ref[...] = jnp.sum(e * e, axis=1)[None, :]
        acc_ref[0] = 0.0

    z = z_ref[...]
    zh = z.astype(jnp.bfloat16)
    z2b = jnp.broadcast_to(jnp.sum(z * z, axis=1, keepdims=True), (BM, 128))
    iota = lax.broadcasted_iota(jnp.int32, (1, 128), 1).astype(jnp.float32)

    runv = jnp.full((BM, 128), jnp.inf, jnp.float32)
    runi = jnp.zeros((BM, 128), jnp.float32)
    for c in range(N_CHUNKS):
        eh_blk = eh_ref[c * BN:(c + 1) * BN, :]
        mm2 = lax.dot_general(zh, eh_blk, (((1,), (1,)), ((), ())),
                              preferred_element_type=jnp.float32)
        for v in range(BN // 128):
            lo = v * 128
            e2v = e2_ref[0, c * BN + lo:c * BN + lo + 128]
            dv = (z2b + e2v[None, :]) + mm2[:, lo:lo + 128]
            iv = iota + float(c * BN + lo)
            m = dv < runv
            runv = jnp.where(m, dv, runv)
            runi = jnp.where(m, jnp.broadcast_to(iv, (BM, 128)), runi)

    lmin = jnp.min(runv, axis=1)
    mini = jnp.min(jnp.where(runv == lmin[:, None], runi, float(NB)), axis=1)
    idx_ref[...] = mini.astype(jnp.int32).reshape(1, 1, BM)
    acc_ref[0] += jnp.sum(lmin)

    @pl.when(pid == GRID_M - 1)
    def _fin():
        loss_ref[0] = acc_ref[0] * ((1.0 + COMMIT) / (BATCH * LATENT))


def _dist_argmin(z, e):
    return pl.pallas_call(
        _dist_argmin_kernel,
        grid=(GRID_M,),
        in_specs=[
            pl.BlockSpec((BM, WORD), lambda i: (i, 0)),
            pl.BlockSpec((NB, WORD), lambda i: (0, 0)),
        ],
        out_specs=[
            pl.BlockSpec((1, 1, BM), lambda i: (i, 0, 0)),
            pl.BlockSpec(memory_space=pltpu.SMEM, block_shape=(1,),
                         index_map=lambda i: (0,)),
        ],
        out_shape=[
            jax.ShapeDtypeStruct((GRID_M, 1, BM), jnp.int32),
            jax.ShapeDtypeStruct((1,), jnp.float32),
        ],
        scratch_shapes=[
            pltpu.VMEM((NB, WORD), jnp.bfloat16),
            pltpu.VMEM((1, NB), jnp.float32),
            pltpu.SMEM((1,), jnp.float32),
        ],
    )(z, e)


_SC_INFO = plsc.get_sparse_core_info()
_NC, _NS = _SC_INFO.num_cores, _SC_INFO.num_subcores
NW = _NC * _NS
BPW = M // NW
CH = 128
NCH = BPW // CH
NBUF = 2


def _gather_body(table_hbm, idx_hbm, out_hbm, idx_v, bufs, sems):
    wid = lax.axis_index("s") * _NC + lax.axis_index("c")
    base = wid * BPW
    pltpu.sync_copy(idx_hbm.at[pl.ds(base, BPW)], idx_v)

    copies = [None] * NBUF
    for b in range(NBUF):
        copies[b] = pltpu.async_copy(
            table_hbm.at[idx_v.at[pl.ds(b * CH, CH)]], bufs.at[b], sems.at[b])
    for c in range(NCH):
        s = c % NBUF
        copies[s].wait()
        pltpu.sync_copy(bufs.at[s], out_hbm.at[pl.ds(base + c * CH, CH)])
        nxt = c + NBUF
        if nxt < NCH:
            copies[s] = pltpu.async_copy(
                table_hbm.at[idx_v.at[pl.ds(nxt * CH, CH)]], bufs.at[s],
                sems.at[s])


def _sc_gather(emb, idx):
    mesh = plsc.VectorSubcoreMesh(core_axis_name="c", subcore_axis_name="s")
    k = functools.partial(
        pl.kernel,
        mesh=mesh,
        out_type=jax.ShapeDtypeStruct((M, WORD), jnp.float32),
        scratch_types=[
            pltpu.VMEM((BPW,), jnp.int32),
            pltpu.VMEM((NBUF, CH, WORD), jnp.float32),
            pltpu.SemaphoreType.DMA((NBUF,)),
        ],
    )(_gather_body)
    return k(emb, idx)


def kernel(inputs, embedding_weight):
    z_mean = inputs[0]
    z = z_mean.reshape(M, WORD)
    idx, loss = _dist_argmin(z, embedding_weight)
    z_q = _sc_gather(embedding_weight, idx.reshape(M))
    return z_q.reshape(z_mean.shape), loss[0]
